# baseline (speedup 1.0000x reference)
import functools

import jax
import jax.numpy as jnp
from jax import lax
from jax.experimental import pallas as pl
from jax.experimental.pallas import tpu as pltpu
from jax.experimental.pallas import tpu_sc as plsc

EMBEDDING_DIM = 32
NUM_CORES = 2
NUM_SUBCORES = 16
NUM_WORKERS = NUM_CORES * NUM_SUBCORES
CHUNK = 512
NBUF = 7


def _make_gather(num_idx: int):
  per_w = num_idx // NUM_WORKERS
  n_chunks = per_w // CHUNK
  assert per_w % CHUNK == 0 and num_idx % NUM_WORKERS == 0

  mesh = plsc.VectorSubcoreMesh(
      core_axis_name="c", subcore_axis_name="s",
      num_cores=NUM_CORES, num_subcores=NUM_SUBCORES)

  @functools.partial(
      pl.kernel,
      mesh=mesh,
      compiler_params=pltpu.CompilerParams(use_tc_tiling_on_sc=False),
      out_type=jax.ShapeDtypeStruct((num_idx, EMBEDDING_DIM), jnp.float32),
      scratch_types=[
          pltpu.VMEM((NBUF, CHUNK), jnp.int32),
          pltpu.VMEM((NBUF, CHUNK, EMBEDDING_DIM), jnp.float32),
          pltpu.SemaphoreType.DMA,
          pltpu.SemaphoreType.DMA,
      ],
  )
  def gather_kernel(idx_hbm, tab_hbm, out_hbm, idx_v, rows_v, sem_g, sem_o):
    wid = lax.axis_index("s") * NUM_CORES + lax.axis_index("c")
    base = wid * per_w

    gathers = [None] * n_chunks
    stores = [None] * n_chunks

    def store_chunk(g):
      gathers[g].wait()
      stores[g] = pltpu.async_copy(
          rows_v.at[g % NBUF], out_hbm.at[pl.ds(base + g * CHUNK, CHUNK)],
          sem_o)

    for g in range(n_chunks):
      b = g % NBUF
      if g >= NBUF:
        stores[g - NBUF].wait()
      pltpu.sync_copy(idx_hbm.at[pl.ds(base + g * CHUNK, CHUNK)],
                      idx_v.at[b])
      gathers[g] = pltpu.async_copy(tab_hbm.at[idx_v.at[b]], rows_v.at[b],
                                    sem_g)
      if g >= NBUF - 1:
        store_chunk(g - NBUF + 1)
    for g in range(max(0, n_chunks - NBUF + 1), n_chunks):
      store_chunk(g)
    for g in range(max(0, n_chunks - NBUF), n_chunks):
      stores[g].wait()

  return gather_kernel


def kernel(input, table):
  batch, fields = input.shape
  num_idx = batch * fields
  idx = input.reshape(num_idx).astype(jnp.int32)
  out = _make_gather(num_idx)(idx, table)
  return out.reshape(batch, fields, EMBEDDING_DIM)
